# Initial kernel scaffold; baseline (speedup 1.0000x reference)
import jax
import jax.numpy as jnp
from jax.experimental import pallas as pl


def kernel(seq, W_word, W_pos, gamma, beta, W_lin, b_lin):
    raise NotImplementedError("write your pallas kernel here")


# speedup vs baseline: 1.4372x; 1.4372x over previous
import functools

import jax
import jax.numpy as jnp
from jax import lax
from jax.experimental import pallas as pl
from jax.experimental.pallas import tpu as pltpu
from jax.experimental.pallas import tpu_sc as plsc

NC = 2
NS = 16
NW = NC * NS

IDX_PER_DMA = 128
DMAS_PER_CHUNK = 5
CHUNK = IDX_PER_DMA * DMAS_PER_CHUNK


def _sc_gather(table, idx3, n_tokens, emb):
    n_per_w = n_tokens // NW
    n_chunks = n_per_w // CHUNK
    mesh = plsc.VectorSubcoreMesh(core_axis_name="c", subcore_axis_name="s")

    @functools.partial(
        pl.kernel,
        mesh=mesh,
        compiler_params=pltpu.CompilerParams(use_tc_tiling_on_sc=False),
        out_type=jax.ShapeDtypeStruct((n_tokens, emb), jnp.float32),
        scratch_types=[
            pltpu.VMEM((n_per_w // IDX_PER_DMA, IDX_PER_DMA), jnp.int32),
            pltpu.VMEM((CHUNK, emb), jnp.float32),
            pltpu.SemaphoreType.DMA,
        ],
    )
    def gather_kernel(table_hbm, idx_hbm, out_hbm, idx_v, rows_v, sem):
        wid = lax.axis_index("s") * NC + lax.axis_index("c")
        base = wid * n_per_w
        pltpu.sync_copy(idx_hbm.at[wid], idx_v)

        def chunk_body(c, carry):
            handles = []
            for j in range(DMAS_PER_CHUNK):
                handles.append(pltpu.async_copy(
                    table_hbm.at[idx_v.at[c * DMAS_PER_CHUNK + j]],
                    rows_v.at[pl.ds(j * IDX_PER_DMA, IDX_PER_DMA)],
                    sem,
                ))
            for h in handles:
                h.wait()
            pltpu.sync_copy(rows_v, out_hbm.at[pl.ds(base + c * CHUNK, CHUNK)])
            return carry

        lax.fori_loop(0, n_chunks, chunk_body, 0)

    return gather_kernel(table, idx3)


def _tc_body(gath_ref, pos_ref, gamma_ref, beta_ref, wlin_ref, blin_ref,
             out_ref):
    x = gath_ref[:] + pos_ref[:]
    mean = jnp.mean(x, axis=1, keepdims=True)
    xc = x - mean
    var = jnp.mean(xc * xc, axis=1, keepdims=True)
    xn = xc * lax.rsqrt(var + 1e-5)
    xn = xn * gamma_ref[:] + beta_ref[:]
    out_ref[:] = (
        jnp.dot(xn, wlin_ref[:], preferred_element_type=jnp.float32)
        + blin_ref[:]
    )


def kernel(seq, W_word, W_pos, gamma, beta, W_lin, b_lin):
    B, L = seq.shape
    emb = W_word.shape[1]
    hid = W_lin.shape[1]
    n = B * L

    n_per_w = n // NW
    idx3 = seq.astype(jnp.int32).reshape(NW, n_per_w // IDX_PER_DMA,
                                         IDX_PER_DMA)
    gathered = _sc_gather(W_word, idx3, n, emb)

    seqs_per_blk = 16
    tblk = seqs_per_blk * L
    grid = n // tblk
    pos_tiled = jnp.tile(W_pos[:L], (seqs_per_blk, 1))

    out = pl.pallas_call(
        _tc_body,
        grid=(grid,),
        in_specs=[
            pl.BlockSpec((tblk, emb), lambda g: (g, 0)),
            pl.BlockSpec((tblk, emb), lambda g: (0, 0)),
            pl.BlockSpec((1, emb), lambda g: (0, 0)),
            pl.BlockSpec((1, emb), lambda g: (0, 0)),
            pl.BlockSpec((emb, hid), lambda g: (0, 0)),
            pl.BlockSpec((1, hid), lambda g: (0, 0)),
        ],
        out_specs=pl.BlockSpec((tblk, hid), lambda g: (g, 0)),
        out_shape=jax.ShapeDtypeStruct((n, hid), jnp.float32),
    )(gathered, pos_tiled, gamma.reshape(1, emb), beta.reshape(1, emb),
      W_lin, b_lin.reshape(1, hid))
    return out.reshape(B, L, hid)


# speedup vs baseline: 1.5826x; 1.1011x over previous
import functools

import jax
import jax.numpy as jnp
from jax import lax
from jax.experimental import pallas as pl
from jax.experimental.pallas import tpu as pltpu
from jax.experimental.pallas import tpu_sc as plsc

NC = 2
NS = 16
NW = NC * NS

IDX_PER_DMA = 128
DMAS_PER_CHUNK = 5
CHUNK = IDX_PER_DMA * DMAS_PER_CHUNK


def _sc_gather(table, idx3, n_tokens, emb, out_minor):
    n_per_w = n_tokens // NW
    n_chunks = n_per_w // CHUNK
    mesh = plsc.VectorSubcoreMesh(core_axis_name="c", subcore_axis_name="s")

    @functools.partial(
        pl.kernel,
        mesh=mesh,
        compiler_params=pltpu.CompilerParams(use_tc_tiling_on_sc=False),
        out_type=jax.ShapeDtypeStruct((n_tokens, out_minor), jnp.float32),
        scratch_types=[
            pltpu.VMEM((n_per_w // IDX_PER_DMA, IDX_PER_DMA), jnp.int32),
            pltpu.VMEM((CHUNK, emb), jnp.float32),
            pltpu.SemaphoreType.DMA,
        ],
    )
    def gather_kernel(table_hbm, idx_hbm, out_hbm, idx_v, rows_v, sem):
        wid = lax.axis_index("s") * NC + lax.axis_index("c")
        base = wid * n_per_w
        pltpu.sync_copy(idx_hbm.at[wid], idx_v)

        def chunk_body(c, carry):
            handles = []
            for j in range(DMAS_PER_CHUNK):
                handles.append(pltpu.async_copy(
                    table_hbm.at[idx_v.at[c * DMAS_PER_CHUNK + j]],
                    rows_v.at[pl.ds(j * IDX_PER_DMA, IDX_PER_DMA)],
                    sem,
                ))
            for h in handles:
                h.wait()
            pltpu.sync_copy(
                rows_v,
                out_hbm.at[pl.ds(base + c * CHUNK, CHUNK), pl.ds(0, emb)])
            return carry

        lax.fori_loop(0, n_chunks, chunk_body, 0)

    return gather_kernel(table, idx3)


def _tc_body(gath_ref, pos_ref, gamma_ref, beta_ref, wlin_ref, blin_ref,
             out_ref):
    x = gath_ref[:, : pos_ref.shape[1]] + pos_ref[:]
    mean = jnp.mean(x, axis=1, keepdims=True)
    xc = x - mean
    var = jnp.mean(xc * xc, axis=1, keepdims=True)
    xn = xc * lax.rsqrt(var + 1e-5)
    xn = xn * gamma_ref[:] + beta_ref[:]
    out_ref[:] = (
        jnp.dot(xn, wlin_ref[:], preferred_element_type=jnp.float32)
        + blin_ref[:]
    )


def kernel(seq, W_word, W_pos, gamma, beta, W_lin, b_lin):
    B, L = seq.shape
    emb = W_word.shape[1]
    hid = W_lin.shape[1]
    n = B * L

    n_per_w = n // NW
    idx3 = seq.astype(jnp.int32).reshape(NW, n_per_w // IDX_PER_DMA,
                                         IDX_PER_DMA)
    gathered = _sc_gather(W_word, idx3, n, emb, 2 * emb)

    seqs_per_blk = 16
    tblk = seqs_per_blk * L
    grid = n // tblk
    pos_tiled = jnp.tile(W_pos[:L], (seqs_per_blk, 1))

    out = pl.pallas_call(
        _tc_body,
        grid=(grid,),
        in_specs=[
            pl.BlockSpec((tblk, 2 * emb), lambda g: (g, 0)),
            pl.BlockSpec((tblk, emb), lambda g: (0, 0)),
            pl.BlockSpec((1, emb), lambda g: (0, 0)),
            pl.BlockSpec((1, emb), lambda g: (0, 0)),
            pl.BlockSpec((emb, hid), lambda g: (0, 0)),
            pl.BlockSpec((1, hid), lambda g: (0, 0)),
        ],
        out_specs=pl.BlockSpec((tblk, hid), lambda g: (g, 0)),
        out_shape=jax.ShapeDtypeStruct((n, hid), jnp.float32),
    )(gathered, pos_tiled, gamma.reshape(1, emb), beta.reshape(1, emb),
      W_lin, b_lin.reshape(1, hid))
    return out.reshape(B, L, hid)


# speedup vs baseline: 2.4453x; 1.5452x over previous
import functools

import jax
import jax.numpy as jnp
from jax import lax
from jax.experimental import pallas as pl
from jax.experimental.pallas import tpu as pltpu
from jax.experimental.pallas import tpu_sc as plsc

NC = 2
NS = 16
NW = NC * NS

IDX_PER_DMA = 128
DMAS_PER_CHUNK = 5
CHUNK = IDX_PER_DMA * DMAS_PER_CHUNK


def _sc_gather(table, idx3, n_tokens, emb, out_minor):
    n_per_w = n_tokens // NW
    n_chunks = n_per_w // CHUNK
    mesh = plsc.VectorSubcoreMesh(core_axis_name="c", subcore_axis_name="s")

    @functools.partial(
        pl.kernel,
        mesh=mesh,
        compiler_params=pltpu.CompilerParams(use_tc_tiling_on_sc=False),
        out_type=jax.ShapeDtypeStruct((n_tokens, out_minor), jnp.float32),
        scratch_types=[
            pltpu.VMEM((n_per_w // IDX_PER_DMA, IDX_PER_DMA), jnp.int32),
            pltpu.VMEM((CHUNK, emb), jnp.float32),
            pltpu.SemaphoreType.DMA,
        ],
    )
    def gather_kernel(table_hbm, idx_hbm, out_hbm, idx_v, rows_v, sem):
        wid = lax.axis_index("s") * NC + lax.axis_index("c")
        base = wid * n_per_w
        pltpu.sync_copy(idx_hbm.at[wid], idx_v)

        def chunk_body(c, carry):
            handles = []
            for j in range(DMAS_PER_CHUNK):
                handles.append(pltpu.async_copy(
                    table_hbm.at[idx_v.at[c * DMAS_PER_CHUNK + j]],
                    rows_v.at[pl.ds(j * IDX_PER_DMA, IDX_PER_DMA)],
                    sem,
                ))
            for h in handles:
                h.wait()
            pltpu.sync_copy(
                rows_v,
                out_hbm.at[pl.ds(base + c * CHUNK, CHUNK), pl.ds(0, emb)])
            return carry

        lax.fori_loop(0, n_chunks, chunk_body, 0)

    return gather_kernel(table, idx3)


TCOL = 4096


def _tc_transpose_pack(Wt):
    emb, V = Wt.shape
    G = -(-V // TCOL)
    half = TCOL // 2

    def body(in_ref, out_ref):
        x = in_ref[:]
        eye = (lax.broadcasted_iota(jnp.int32, (emb, emb), 0)
               == lax.broadcasted_iota(jnp.int32, (emb, emb), 1)
               ).astype(jnp.float32)
        xt = lax.dot_general(x, eye, (((0,), (0,)), ((), ())),
                             preferred_element_type=jnp.float32)
        out_ref[:] = jnp.concatenate([xt[:half], xt[half:]], axis=1)

    return pl.pallas_call(
        body,
        grid=(G,),
        in_specs=[pl.BlockSpec((emb, TCOL), lambda g: (0, g))],
        out_specs=pl.BlockSpec((half, 2 * emb), lambda g: (g, 0)),
        out_shape=jax.ShapeDtypeStruct((G * half, 2 * emb), jnp.float32),
    )(Wt)


def _tc_body(gath_ref, pos_ref, gamma_ref, beta_ref, wlin_ref, blin_ref,
             out_ref):
    x = gath_ref[:, : pos_ref.shape[1]] + pos_ref[:]
    mean = jnp.mean(x, axis=1, keepdims=True)
    xc = x - mean
    var = jnp.mean(xc * xc, axis=1, keepdims=True)
    xn = xc * lax.rsqrt(var + 1e-5)
    xn = xn * gamma_ref[:] + beta_ref[:]
    out_ref[:] = (
        jnp.dot(xn, wlin_ref[:], preferred_element_type=jnp.float32)
        + blin_ref[:]
    )


def kernel(seq, W_word, W_pos, gamma, beta, W_lin, b_lin):
    B, L = seq.shape
    emb = W_word.shape[1]
    hid = W_lin.shape[1]
    n = B * L

    packed = _tc_transpose_pack(W_word.T)
    table_lin = packed.reshape(-1, emb)

    n_per_w = n // NW
    v = seq.astype(jnp.int32).reshape(-1)
    g = v // TCOL
    u = v - g * TCOL
    half = TCOL // 2
    lin = g * TCOL + 2 * (u % half) + u // half
    idx3 = lin.reshape(NW, n_per_w // IDX_PER_DMA, IDX_PER_DMA)
    gathered = _sc_gather(table_lin, idx3, n, emb, 2 * emb)

    seqs_per_blk = 16
    tblk = seqs_per_blk * L
    grid = n // tblk
    pos_tiled = jnp.tile(W_pos[:L], (seqs_per_blk, 1))

    out = pl.pallas_call(
        _tc_body,
        grid=(grid,),
        in_specs=[
            pl.BlockSpec((tblk, 2 * emb), lambda g: (g, 0)),
            pl.BlockSpec((tblk, emb), lambda g: (0, 0)),
            pl.BlockSpec((1, emb), lambda g: (0, 0)),
            pl.BlockSpec((1, emb), lambda g: (0, 0)),
            pl.BlockSpec((emb, hid), lambda g: (0, 0)),
            pl.BlockSpec((1, hid), lambda g: (0, 0)),
        ],
        out_specs=pl.BlockSpec((tblk, hid), lambda g: (g, 0)),
        out_shape=jax.ShapeDtypeStruct((n, hid), jnp.float32),
    )(gathered, pos_tiled, gamma.reshape(1, emb), beta.reshape(1, emb),
      W_lin, b_lin.reshape(1, hid))
    return out.reshape(B, L, hid)


# speedup vs baseline: 2.8451x; 1.1635x over previous
import functools

import jax
import jax.numpy as jnp
from jax import lax
from jax.experimental import pallas as pl
from jax.experimental.pallas import tpu as pltpu
from jax.experimental.pallas import tpu_sc as plsc

NC = 2
NS = 16
NW = NC * NS

IDX_PER_DMA = 128
DMAS_PER_CHUNK = 5
CHUNK = IDX_PER_DMA * DMAS_PER_CHUNK


def _sc_gather(table, idx3, n_tokens, emb, out_minor):
    n_per_w = n_tokens // NW
    n_chunks = n_per_w // CHUNK
    mesh = plsc.VectorSubcoreMesh(core_axis_name="c", subcore_axis_name="s")

    @functools.partial(
        pl.kernel,
        mesh=mesh,
        compiler_params=pltpu.CompilerParams(use_tc_tiling_on_sc=False),
        out_type=jax.ShapeDtypeStruct((n_tokens, out_minor), jnp.float32),
        scratch_types=[
            pltpu.VMEM((n_per_w // IDX_PER_DMA, IDX_PER_DMA), jnp.int32),
            pltpu.VMEM((CHUNK, emb), jnp.float32),
            pltpu.SemaphoreType.DMA,
        ],
    )
    def gather_kernel(table_hbm, idx_hbm, out_hbm, idx_v, rows_v, sem):
        wid = lax.axis_index("s") * NC + lax.axis_index("c")
        base = wid * n_per_w
        pltpu.sync_copy(idx_hbm.at[wid], idx_v)

        def chunk_body(c, carry):
            handles = []
            for j in range(DMAS_PER_CHUNK):
                handles.append(pltpu.async_copy(
                    table_hbm.at[idx_v.at[c * DMAS_PER_CHUNK + j]],
                    rows_v.at[pl.ds(j * IDX_PER_DMA, IDX_PER_DMA)],
                    sem,
                ))
            for h in handles:
                h.wait()
            pltpu.sync_copy(
                rows_v,
                out_hbm.at[pl.ds(base + c * CHUNK, CHUNK), pl.ds(0, emb)])
            return carry

        lax.fori_loop(0, n_chunks, chunk_body, 0)

    return gather_kernel(table, idx3)


TCOL = 8192


def _tc_transpose_pack(Wt):
    emb, V = Wt.shape
    G = -(-V // TCOL)
    half = TCOL // 2

    def body(in_ref, out_ref):
        x = in_ref[:]
        xt = jnp.transpose(x, (1, 0))
        out_ref[:] = jnp.concatenate([xt[:half], xt[half:]], axis=1)

    return pl.pallas_call(
        body,
        grid=(G,),
        in_specs=[pl.BlockSpec((emb, TCOL), lambda g: (0, g))],
        out_specs=pl.BlockSpec((half, 2 * emb), lambda g: (g, 0)),
        out_shape=jax.ShapeDtypeStruct((G * half, 2 * emb), jnp.float32),
    )(Wt)


def _tc_body(gath_ref, pos_ref, gamma_ref, beta_ref, wlin_ref, blin_ref,
             out_ref):
    x = gath_ref[:, : pos_ref.shape[1]] + pos_ref[:]
    mean = jnp.mean(x, axis=1, keepdims=True)
    xc = x - mean
    var = jnp.mean(xc * xc, axis=1, keepdims=True)
    xn = xc * lax.rsqrt(var + 1e-5)
    xn = xn * gamma_ref[:] + beta_ref[:]
    out_ref[:] = (
        jnp.dot(xn, wlin_ref[:], preferred_element_type=jnp.float32)
        + blin_ref[:]
    )


def kernel(seq, W_word, W_pos, gamma, beta, W_lin, b_lin):
    B, L = seq.shape
    emb = W_word.shape[1]
    hid = W_lin.shape[1]
    n = B * L

    packed = _tc_transpose_pack(W_word.T)
    table_lin = packed.reshape(-1, emb)

    n_per_w = n // NW
    v = seq.astype(jnp.int32).reshape(-1)
    g = v // TCOL
    u = v - g * TCOL
    half = TCOL // 2
    lin = g * TCOL + 2 * (u % half) + u // half
    idx3 = lin.reshape(NW, n_per_w // IDX_PER_DMA, IDX_PER_DMA)
    gathered = _sc_gather(table_lin, idx3, n, emb, 2 * emb)

    seqs_per_blk = 16
    tblk = seqs_per_blk * L
    grid = n // tblk
    pos_tiled = jnp.tile(W_pos[:L], (seqs_per_blk, 1))

    out = pl.pallas_call(
        _tc_body,
        grid=(grid,),
        in_specs=[
            pl.BlockSpec((tblk, 2 * emb), lambda g: (g, 0)),
            pl.BlockSpec((tblk, emb), lambda g: (0, 0)),
            pl.BlockSpec((1, emb), lambda g: (0, 0)),
            pl.BlockSpec((1, emb), lambda g: (0, 0)),
            pl.BlockSpec((emb, hid), lambda g: (0, 0)),
            pl.BlockSpec((1, hid), lambda g: (0, 0)),
        ],
        out_specs=pl.BlockSpec((tblk, hid), lambda g: (g, 0)),
        out_shape=jax.ShapeDtypeStruct((n, hid), jnp.float32),
    )(gathered, pos_tiled, gamma.reshape(1, emb), beta.reshape(1, emb),
      W_lin, b_lin.reshape(1, hid))
    return out.reshape(B, L, hid)


# speedup vs baseline: 3.2598x; 1.1458x over previous
import functools

import jax
import jax.numpy as jnp
from jax import lax
from jax.experimental import pallas as pl
from jax.experimental.pallas import tpu as pltpu
from jax.experimental.pallas import tpu_sc as plsc

NC = 2
NS = 16
NW = NC * NS

IDX_PER_DMA = 128
DMAS_PER_CHUNK = 5
CHUNK = IDX_PER_DMA * DMAS_PER_CHUNK

TCOL = 16384


def _tc_transpose_pack(Wt):
    emb, V = Wt.shape
    G = -(-V // TCOL)
    half = TCOL // 2

    def body(in_ref, out_ref):
        x = in_ref[:]
        xt = jnp.transpose(x, (1, 0))
        out_ref[:] = jnp.concatenate([xt[:half], xt[half:]], axis=1)

    return pl.pallas_call(
        body,
        grid=(G,),
        in_specs=[pl.BlockSpec((emb, TCOL), lambda g: (0, g))],
        out_specs=pl.BlockSpec((half, 2 * emb), lambda g: (g, 0)),
        out_shape=jax.ShapeDtypeStruct((G * half, 2 * emb), jnp.float32),
    )(Wt)


def _sc_gather(table, idx3, n_tokens, emb, out_minor):
    n_per_w = n_tokens // NW
    n_rows = n_per_w // IDX_PER_DMA
    n_chunks = n_per_w // CHUNK
    half = TCOL // 2
    mesh = plsc.VectorSubcoreMesh(core_axis_name="c", subcore_axis_name="s")

    @functools.partial(
        pl.kernel,
        mesh=mesh,
        compiler_params=pltpu.CompilerParams(use_tc_tiling_on_sc=False),
        out_type=jax.ShapeDtypeStruct((n_tokens, out_minor), jnp.float32),
        scratch_types=[
            pltpu.VMEM((n_rows, IDX_PER_DMA), jnp.int32),
            pltpu.VMEM((n_rows, IDX_PER_DMA), jnp.int32),
            pltpu.VMEM((CHUNK, emb), jnp.float32),
            pltpu.SemaphoreType.DMA,
        ],
    )
    def gather_kernel(table_hbm, idx_hbm, out_hbm, idx_v, lin_v, rows_v, sem):
        wid = lax.axis_index("s") * NC + lax.axis_index("c")
        base = wid * n_per_w
        pltpu.sync_copy(idx_hbm.at[wid], idx_v)

        def remap_row(r, carry):
            for j in range(IDX_PER_DMA // 16):
                v = idx_v[r, pl.ds(j * 16, 16)]
                u = jnp.bitwise_and(v, TCOL - 1)
                lin = (jnp.bitwise_and(v, -TCOL)
                       + lax.shift_left(jnp.bitwise_and(u, half - 1), 1)
                       + lax.shift_right_logical(u, half.bit_length() - 1))
                lin_v[r, pl.ds(j * 16, 16)] = lin
            return carry

        lax.fori_loop(0, n_rows, remap_row, 0)

        def chunk_body(c, carry):
            handles = []
            for j in range(DMAS_PER_CHUNK):
                handles.append(pltpu.async_copy(
                    table_hbm.at[lin_v.at[c * DMAS_PER_CHUNK + j]],
                    rows_v.at[pl.ds(j * IDX_PER_DMA, IDX_PER_DMA)],
                    sem,
                ))
            for h in handles:
                h.wait()
            pltpu.sync_copy(
                rows_v,
                out_hbm.at[pl.ds(base + c * CHUNK, CHUNK), pl.ds(0, emb)])
            return carry

        lax.fori_loop(0, n_chunks, chunk_body, 0)

    return gather_kernel(table, idx3)


def _tc_body(gath_ref, pos_ref, gamma_ref, beta_ref, wlin_ref, blin_ref,
             out_ref):
    L = pos_ref.shape[0]
    emb = pos_ref.shape[1]
    tblk = gath_ref.shape[0]
    x = gath_ref[:, :emb]
    pos = pos_ref[:]
    x = jnp.concatenate(
        [x[s * L:(s + 1) * L, :] + pos for s in range(tblk // L)], axis=0)
    mean = jnp.mean(x, axis=1, keepdims=True)
    xc = x - mean
    var = jnp.mean(xc * xc, axis=1, keepdims=True)
    xn = xc * lax.rsqrt(var + 1e-5)
    xn = xn * gamma_ref[:] + beta_ref[:]
    out_ref[:] = (
        jnp.dot(xn, wlin_ref[:], preferred_element_type=jnp.float32)
        + blin_ref[:]
    )


def kernel(seq, W_word, W_pos, gamma, beta, W_lin, b_lin):
    B, L = seq.shape
    emb = W_word.shape[1]
    hid = W_lin.shape[1]
    n = B * L

    packed = _tc_transpose_pack(W_word.T)
    table_lin = packed.reshape(-1, emb)

    n_per_w = n // NW
    idx3 = seq.astype(jnp.int32).reshape(NW, n_per_w // IDX_PER_DMA,
                                         IDX_PER_DMA)
    gathered = _sc_gather(table_lin, idx3, n, emb, 2 * emb)

    seqs_per_blk = 32
    tblk = seqs_per_blk * L
    grid = n // tblk

    out = pl.pallas_call(
        _tc_body,
        grid=(grid,),
        in_specs=[
            pl.BlockSpec((tblk, 2 * emb), lambda g: (g, 0)),
            pl.BlockSpec((L, emb), lambda g: (0, 0)),
            pl.BlockSpec((1, emb), lambda g: (0, 0)),
            pl.BlockSpec((1, emb), lambda g: (0, 0)),
            pl.BlockSpec((emb, hid), lambda g: (0, 0)),
            pl.BlockSpec((1, hid), lambda g: (0, 0)),
        ],
        out_specs=pl.BlockSpec((tblk, hid), lambda g: (g, 0)),
        out_shape=jax.ShapeDtypeStruct((n, hid), jnp.float32),
    )(gathered, W_pos, gamma.reshape(1, emb), beta.reshape(1, emb),
      W_lin, b_lin.reshape(1, hid))
    return out.reshape(B, L, hid)


# speedup vs baseline: 3.4001x; 1.0430x over previous
import functools

import jax
import jax.numpy as jnp
from jax import lax
from jax.experimental import pallas as pl
from jax.experimental.pallas import tpu as pltpu
from jax.experimental.pallas import tpu_sc as plsc

NC = 2
NS = 16
NW = NC * NS

IDX_PER_DMA = 128
DMAS_PER_CHUNK = 5
CHUNK = IDX_PER_DMA * DMAS_PER_CHUNK

TCOL = 32768


def _tc_transpose_pack(Wt):
    emb, V = Wt.shape
    G = -(-V // TCOL)
    half = TCOL // 2

    def body(in_ref, out_ref):
        x = in_ref[:]
        xt = jnp.transpose(x, (1, 0))
        out_ref[:, :emb] = xt[:half]
        out_ref[:, emb:] = xt[half:]

    return pl.pallas_call(
        body,
        grid=(G,),
        in_specs=[pl.BlockSpec((emb, TCOL), lambda g: (0, g))],
        out_specs=pl.BlockSpec((half, 2 * emb), lambda g: (g, 0)),
        out_shape=jax.ShapeDtypeStruct((G * half, 2 * emb), jnp.float32),
    )(Wt)


def _sc_gather(table, idx3, n_tokens, emb, out_minor):
    n_per_w = n_tokens // NW
    n_rows = n_per_w // IDX_PER_DMA
    n_chunks = n_per_w // CHUNK
    half = TCOL // 2
    mesh = plsc.VectorSubcoreMesh(core_axis_name="c", subcore_axis_name="s")

    @functools.partial(
        pl.kernel,
        mesh=mesh,
        compiler_params=pltpu.CompilerParams(use_tc_tiling_on_sc=False),
        out_type=jax.ShapeDtypeStruct((n_tokens, out_minor), jnp.float32),
        scratch_types=[
            pltpu.VMEM((n_rows, IDX_PER_DMA), jnp.int32),
            pltpu.VMEM((n_rows, IDX_PER_DMA), jnp.int32),
            pltpu.VMEM((CHUNK, emb), jnp.float32),
            pltpu.SemaphoreType.DMA,
        ],
    )
    def gather_kernel(table_hbm, idx_hbm, out_hbm, idx_v, lin_v, rows_v, sem):
        wid = lax.axis_index("s") * NC + lax.axis_index("c")
        base = wid * n_per_w
        pltpu.sync_copy(idx_hbm.at[wid], idx_v)

        def remap_row(r, carry):
            for j in range(IDX_PER_DMA // 16):
                v = idx_v[r, pl.ds(j * 16, 16)]
                u = jnp.bitwise_and(v, TCOL - 1)
                lin = (jnp.bitwise_and(v, -TCOL)
                       + lax.shift_left(jnp.bitwise_and(u, half - 1), 1)
                       + lax.shift_right_logical(u, half.bit_length() - 1))
                lin_v[r, pl.ds(j * 16, 16)] = lin
            return carry

        lax.fori_loop(0, n_rows, remap_row, 0)

        def chunk_body(c, carry):
            handles = []
            for j in range(DMAS_PER_CHUNK):
                handles.append(pltpu.async_copy(
                    table_hbm.at[lin_v.at[c * DMAS_PER_CHUNK + j]],
                    rows_v.at[pl.ds(j * IDX_PER_DMA, IDX_PER_DMA)],
                    sem,
                ))
            for h in handles:
                h.wait()
            pltpu.sync_copy(
                rows_v,
                out_hbm.at[pl.ds(base + c * CHUNK, CHUNK), pl.ds(0, emb)])
            return carry

        lax.fori_loop(0, n_chunks, chunk_body, 0)

    return gather_kernel(table, idx3)


def _tc_body(gath_ref, pos_ref, gamma_ref, beta_ref, wlin_ref, blin_ref,
             out_ref):
    L = pos_ref.shape[0]
    emb = pos_ref.shape[1]
    tblk = gath_ref.shape[0]
    x = gath_ref[:, :emb]
    pos = pos_ref[:]
    x = jnp.concatenate(
        [x[s * L:(s + 1) * L, :] + pos for s in range(tblk // L)], axis=0)
    mean = jnp.mean(x, axis=1, keepdims=True)
    xc = x - mean
    var = jnp.mean(xc * xc, axis=1, keepdims=True)
    xn = xc * lax.rsqrt(var + 1e-5)
    xn = xn * gamma_ref[:] + beta_ref[:]
    out_ref[:] = (
        jnp.dot(xn, wlin_ref[:], preferred_element_type=jnp.float32)
        + blin_ref[:]
    )


def kernel(seq, W_word, W_pos, gamma, beta, W_lin, b_lin):
    B, L = seq.shape
    emb = W_word.shape[1]
    hid = W_lin.shape[1]
    n = B * L

    packed = _tc_transpose_pack(W_word.T)
    table_lin = packed.reshape(-1, emb)

    n_per_w = n // NW
    idx3 = seq.astype(jnp.int32).reshape(NW, n_per_w // IDX_PER_DMA,
                                         IDX_PER_DMA)
    gathered = _sc_gather(table_lin, idx3, n, emb, 2 * emb)

    seqs_per_blk = 32
    tblk = seqs_per_blk * L
    grid = n // tblk

    out = pl.pallas_call(
        _tc_body,
        grid=(grid,),
        in_specs=[
            pl.BlockSpec((tblk, 2 * emb), lambda g: (g, 0)),
            pl.BlockSpec((L, emb), lambda g: (0, 0)),
            pl.BlockSpec((1, emb), lambda g: (0, 0)),
            pl.BlockSpec((1, emb), lambda g: (0, 0)),
            pl.BlockSpec((emb, hid), lambda g: (0, 0)),
            pl.BlockSpec((1, hid), lambda g: (0, 0)),
        ],
        out_specs=pl.BlockSpec((tblk, hid), lambda g: (g, 0)),
        out_shape=jax.ShapeDtypeStruct((n, hid), jnp.float32),
    )(gathered, W_pos, gamma.reshape(1, emb), beta.reshape(1, emb),
      W_lin, b_lin.reshape(1, hid))
    return out.reshape(B, L, hid)


# speedup vs baseline: 3.4442x; 1.0130x over previous
import functools

import jax
import jax.numpy as jnp
from jax import lax
from jax.experimental import pallas as pl
from jax.experimental.pallas import tpu as pltpu
from jax.experimental.pallas import tpu_sc as plsc

NC = 2
NS = 16
NW = NC * NS

IDX_PER_DMA = 128
DMAS_PER_CHUNK = 10
CHUNK = IDX_PER_DMA * DMAS_PER_CHUNK

TCOL = 32768


def _tc_transpose_pack(Wt):
    emb, V = Wt.shape
    G = -(-V // TCOL)
    half = TCOL // 2

    def body(in_ref, out_ref):
        x = in_ref[:]
        xt = jnp.transpose(x, (1, 0))
        out_ref[:, :emb] = xt[:half]
        out_ref[:, emb:] = xt[half:]

    return pl.pallas_call(
        body,
        grid=(G,),
        in_specs=[pl.BlockSpec((emb, TCOL), lambda g: (0, g))],
        out_specs=pl.BlockSpec((half, 2 * emb), lambda g: (g, 0)),
        out_shape=jax.ShapeDtypeStruct((G * half, 2 * emb), jnp.float32),
    )(Wt)


def _sc_gather(table, idx3, n_tokens, emb, out_minor):
    n_per_w = n_tokens // NW
    n_rows = n_per_w // IDX_PER_DMA
    n_chunks = n_per_w // CHUNK
    half = TCOL // 2
    mesh = plsc.VectorSubcoreMesh(core_axis_name="c", subcore_axis_name="s")

    @functools.partial(
        pl.kernel,
        mesh=mesh,
        compiler_params=pltpu.CompilerParams(use_tc_tiling_on_sc=False),
        out_type=jax.ShapeDtypeStruct((n_tokens, out_minor), jnp.float32),
        scratch_types=[
            pltpu.VMEM((n_rows, IDX_PER_DMA), jnp.int32),
            pltpu.VMEM((n_rows, IDX_PER_DMA), jnp.int32),
            pltpu.VMEM((CHUNK, emb), jnp.float32),
            pltpu.SemaphoreType.DMA,
        ],
    )
    def gather_kernel(table_hbm, idx_hbm, out_hbm, idx_v, lin_v, rows_v, sem):
        wid = lax.axis_index("s") * NC + lax.axis_index("c")
        base = wid * n_per_w
        pltpu.sync_copy(idx_hbm.at[wid], idx_v)

        def remap_row(r, carry):
            for j in range(IDX_PER_DMA // 16):
                v = idx_v[r, pl.ds(j * 16, 16)]
                u = jnp.bitwise_and(v, TCOL - 1)
                lin = (jnp.bitwise_and(v, -TCOL)
                       + lax.shift_left(jnp.bitwise_and(u, half - 1), 1)
                       + lax.shift_right_logical(u, half.bit_length() - 1))
                lin_v[r, pl.ds(j * 16, 16)] = lin
            return carry

        lax.fori_loop(0, n_rows, remap_row, 0)

        def chunk_body(c, carry):
            handles = []
            for j in range(DMAS_PER_CHUNK):
                handles.append(pltpu.async_copy(
                    table_hbm.at[lin_v.at[c * DMAS_PER_CHUNK + j]],
                    rows_v.at[pl.ds(j * IDX_PER_DMA, IDX_PER_DMA)],
                    sem,
                ))
            for h in handles:
                h.wait()
            pltpu.sync_copy(
                rows_v,
                out_hbm.at[pl.ds(base + c * CHUNK, CHUNK), pl.ds(0, emb)])
            return carry

        lax.fori_loop(0, n_chunks, chunk_body, 0)

    return gather_kernel(table, idx3)


def _tc_body(gath_ref, pos_ref, gamma_ref, beta_ref, wlin_ref, blin_ref,
             out_ref):
    L = pos_ref.shape[0]
    emb = pos_ref.shape[1]
    tblk = gath_ref.shape[0]
    x = gath_ref[:, :emb]
    pos = pos_ref[:]
    x = jnp.concatenate(
        [x[s * L:(s + 1) * L, :] + pos for s in range(tblk // L)], axis=0)
    mean = jnp.mean(x, axis=1, keepdims=True)
    xc = x - mean
    var = jnp.mean(xc * xc, axis=1, keepdims=True)
    xn = xc * lax.rsqrt(var + 1e-5)
    xn = xn * gamma_ref[:] + beta_ref[:]
    out_ref[:] = (
        jnp.dot(xn, wlin_ref[:], preferred_element_type=jnp.float32)
        + blin_ref[:]
    )


def kernel(seq, W_word, W_pos, gamma, beta, W_lin, b_lin):
    B, L = seq.shape
    emb = W_word.shape[1]
    hid = W_lin.shape[1]
    n = B * L

    packed = _tc_transpose_pack(W_word.T)
    table_lin = packed.reshape(-1, emb)

    n_per_w = n // NW
    idx3 = seq.astype(jnp.int32).reshape(NW, n_per_w // IDX_PER_DMA,
                                         IDX_PER_DMA)
    gathered = _sc_gather(table_lin, idx3, n, emb, 2 * emb)

    seqs_per_blk = 32
    tblk = seqs_per_blk * L
    grid = n // tblk

    out = pl.pallas_call(
        _tc_body,
        grid=(grid,),
        in_specs=[
            pl.BlockSpec((tblk, 2 * emb), lambda g: (g, 0)),
            pl.BlockSpec((L, emb), lambda g: (0, 0)),
            pl.BlockSpec((1, emb), lambda g: (0, 0)),
            pl.BlockSpec((1, emb), lambda g: (0, 0)),
            pl.BlockSpec((emb, hid), lambda g: (0, 0)),
            pl.BlockSpec((1, hid), lambda g: (0, 0)),
        ],
        out_specs=pl.BlockSpec((tblk, hid), lambda g: (g, 0)),
        out_shape=jax.ShapeDtypeStruct((n, hid), jnp.float32),
    )(gathered, W_pos, gamma.reshape(1, emb), beta.reshape(1, emb),
      W_lin, b_lin.reshape(1, hid))
    return out.reshape(B, L, hid)
